# baseline (speedup 1.0000x reference)
import functools
import itertools

import jax
import jax.numpy as jnp
import numpy as np
from jax import lax
from jax.experimental import pallas as pl
from jax.experimental.pallas import tpu as pltpu
from jax.experimental.pallas import tpu_sc as plsc

_N_BITS = 20
_ORDER = 4
_BATCH = 4096
_NC = 2
_NS = 16
_NW = _NC * _NS
_ROWS_PER_W = _BATCH // _NW
_R = 8
_NCHUNK = _ROWS_PER_W // _R
_XCOLS = 24
_LANES = 16


def _build_tables():
    combos = []
    for size in range(1, _ORDER + 1):
        combos.extend(itertools.combinations(range(_N_BITS), size))
    num_terms = len(combos)
    tpad = ((num_terms + _LANES - 1) // _LANES) * _LANES
    par_x = np.full(tpad, 20, np.int32)
    par_t = np.zeros(tpad, np.int32)
    from_term = np.zeros(tpad, bool)
    leaf = np.full(tpad, 21, np.int32)
    idx_of = {}
    for t, c in enumerate(combos):
        idx_of[c] = t
        leaf[t] = c[-1]
        if len(c) == 1:
            par_x[t] = 20
        elif len(c) == 2:
            par_x[t] = c[0]
        else:
            from_term[t] = True
            par_t[t] = idx_of[c[:-1]]
    for t in range(num_terms):
        if from_term[t]:
            assert par_t[t] // _LANES < t // _LANES
    first_term_parent = int(np.argmax(from_term))
    mixed_chunk = first_term_parent // _LANES
    assert not from_term[: mixed_chunk * _LANES].any()
    mixed_lane = first_term_parent - mixed_chunk * _LANES
    assert from_term[first_term_parent: num_terms].all()
    return (par_x, par_t, leaf, mixed_lane, mixed_chunk, num_terms, tpad)


(_PARX_NP, _PART_NP, _LEAF_NP, _MIXED_LANE, _MIXED_CHUNK, _NUM_TERMS,
 _TPAD) = _build_tables()
_TC_STEPS = _TPAD // _LANES
_SLAB = _R * _NUM_TERMS


def _sc_parity(x_hbm, parx_hbm, part_hbm, leaf_hbm, out_hbm,
               parx_v, part_v, leaf_v, x_v, stage):
    wid = lax.axis_index("s") * _NC + lax.axis_index("c")
    pltpu.sync_copy(parx_hbm, parx_v)
    pltpu.sync_copy(part_hbm, part_v)
    pltpu.sync_copy(leaf_hbm, leaf_v)
    lane = lax.iota(jnp.int32, _LANES)
    mixed = lane >= jnp.int32(_MIXED_LANE)
    tail_mask = lane < jnp.int32(_NUM_TERMS - (_TC_STEPS - 1) * _LANES)

    def slab_body(c, carry):
        base = wid * _ROWS_PER_W + c * _R
        pltpu.sync_copy(x_hbm.at[pl.ds(base * _XCOLS, _R * _XCOLS)], x_v)

        def step(s, parent_of):
            lvec = leaf_v[pl.ds(s * _LANES, _LANES)]
            pvec = parent_of[pl.ds(s * _LANES, _LANES)]
            tcol = s * _LANES + lane
            for r in range(_R):
                pa = (plsc.load_gather(stage, [pvec + jnp.int32(r * _NUM_TERMS)])
                      if parent_of is part_v else
                      plsc.load_gather(x_v, [pvec + jnp.int32(r * _XCOLS)]))
                lf = plsc.load_gather(x_v, [lvec + jnp.int32(r * _XCOLS)])
                plsc.store_scatter(stage, [tcol + jnp.int32(r * _NUM_TERMS)],
                                   pa * lf)
            return parent_of

        lax.fori_loop(0, _MIXED_CHUNK, lambda s, _: (step(s, parx_v), 0)[1], 0)
        s = _MIXED_CHUNK
        lvec = leaf_v[pl.ds(s * _LANES, _LANES)]
        pxv = parx_v[pl.ds(s * _LANES, _LANES)]
        ptv = part_v[pl.ds(s * _LANES, _LANES)]
        tcol = s * _LANES + lane
        for r in range(_R):
            pa_x = plsc.load_gather(x_v, [pxv + jnp.int32(r * _XCOLS)])
            pa_t = plsc.load_gather(stage, [ptv + jnp.int32(r * _NUM_TERMS)])
            lf = plsc.load_gather(x_v, [lvec + jnp.int32(r * _XCOLS)])
            pa = jnp.where(mixed, pa_t, pa_x)
            plsc.store_scatter(stage, [tcol + jnp.int32(r * _NUM_TERMS)],
                               pa * lf)
        lax.fori_loop(_MIXED_CHUNK + 1, _TC_STEPS - 1,
                      lambda s2, _: (step(s2, part_v), 0)[1], 0)
        s = _TC_STEPS - 1
        lvec = leaf_v[pl.ds(s * _LANES, _LANES)]
        ptv = part_v[pl.ds(s * _LANES, _LANES)]
        tcol = s * _LANES + lane
        for r in range(_R):
            pa = plsc.load_gather(stage, [ptv + jnp.int32(r * _NUM_TERMS)])
            lf = plsc.load_gather(x_v, [lvec + jnp.int32(r * _XCOLS)])
            plsc.store_scatter(stage, [tcol + jnp.int32(r * _NUM_TERMS)],
                               pa * lf, mask=tail_mask)
        pltpu.sync_copy(stage, out_hbm.at[pl.ds(base * _NUM_TERMS, _SLAB)])
        return carry

    lax.fori_loop(0, _NCHUNK, slab_body, 0)


@functools.lru_cache(maxsize=1)
def _sc_call():
    return functools.partial(
        pl.kernel,
        out_type=jax.ShapeDtypeStruct((_BATCH * _NUM_TERMS,), jnp.float32),
        mesh=plsc.VectorSubcoreMesh(core_axis_name="c", subcore_axis_name="s"),
        scratch_types=[
            pltpu.VMEM((_TPAD,), jnp.int32),
            pltpu.VMEM((_TPAD,), jnp.int32),
            pltpu.VMEM((_TPAD,), jnp.int32),
            pltpu.VMEM((_R * _XCOLS,), jnp.float32),
            pltpu.VMEM((_SLAB,), jnp.float32),
        ],
        compiler_params=pltpu.CompilerParams(
            use_tc_tiling_on_sc=False, needs_layout_passes=False),
    )(_sc_parity)


@jax.jit
def kernel(inputs):
    batch = inputs.shape[0]
    x = jnp.concatenate(
        [inputs,
         jnp.ones((batch, 1), jnp.float32),
         jnp.zeros((batch, _XCOLS - _N_BITS - 1), jnp.float32)],
        axis=1).reshape(-1)
    flat = _sc_call()(x, jnp.asarray(_PARX_NP), jnp.asarray(_PART_NP),
                      jnp.asarray(_LEAF_NP))
    return flat.reshape(batch, _NUM_TERMS)


# speedup vs baseline: 1.2727x; 1.2727x over previous
import functools
import itertools

import jax
import jax.numpy as jnp
import numpy as np
from jax import lax
from jax.experimental import pallas as pl
from jax.experimental.pallas import tpu as pltpu
from jax.experimental.pallas import tpu_sc as plsc

_N_BITS = 20
_ORDER = 4
_BATCH = 4096
_NC = 2
_NS = 16
_NW = _NC * _NS
_ROWS_PER_W = _BATCH // _NW
_R = 8
_NCHUNK = _ROWS_PER_W // _R
_XCOLS = 24
_LANES = 16


def _build_tables():
    combos = []
    for size in range(1, _ORDER + 1):
        combos.extend(itertools.combinations(range(_N_BITS), size))
    num_terms = len(combos)
    tpad = ((num_terms + _LANES - 1) // _LANES) * _LANES
    par_x = np.full(tpad, 20, np.int32)
    par_t = np.zeros(tpad, np.int32)
    from_term = np.zeros(tpad, bool)
    leaf = np.full(tpad, 21, np.int32)
    idx_of = {}
    for t, c in enumerate(combos):
        idx_of[c] = t
        leaf[t] = c[-1]
        if len(c) == 1:
            par_x[t] = 20
        elif len(c) == 2:
            par_x[t] = c[0]
        else:
            from_term[t] = True
            par_t[t] = idx_of[c[:-1]]
    for t in range(num_terms):
        if from_term[t]:
            assert par_t[t] // _LANES < t // _LANES
    first_term_parent = int(np.argmax(from_term))
    mixed_chunk = first_term_parent // _LANES
    assert not from_term[: mixed_chunk * _LANES].any()
    mixed_lane = first_term_parent - mixed_chunk * _LANES
    assert from_term[first_term_parent: num_terms].all()
    return (par_x, par_t, leaf, mixed_lane, mixed_chunk, num_terms, tpad)


(_PARX_NP, _PART_NP, _LEAF_NP, _MIXED_LANE, _MIXED_CHUNK, _NUM_TERMS,
 _TPAD) = _build_tables()
_TC_STEPS = _TPAD // _LANES
_SLAB = _R * _NUM_TERMS
_B_END = 85


def _sc_parity(x_hbm, parx_hbm, part_hbm, leaf_hbm, out_hbm,
               parx_v, part_v, leaf_v, x_all, stage0, stage1, sem0, sem1):
    wid = lax.axis_index("s") * _NC + lax.axis_index("c")
    pltpu.sync_copy(parx_hbm, parx_v)
    pltpu.sync_copy(part_hbm, part_v)
    pltpu.sync_copy(leaf_hbm, leaf_v)
    pltpu.sync_copy(
        x_hbm.at[pl.ds(wid * _ROWS_PER_W * _XCOLS, _ROWS_PER_W * _XCOLS)],
        x_all)
    lane = lax.iota(jnp.int32, _LANES)
    mixed = lane >= jnp.int32(_MIXED_LANE)
    tail_mask = lane < jnp.int32(_NUM_TERMS - (_TC_STEPS - 1) * _LANES)

    def compute_slab(sl, stage):
        xoff = sl * (_R * _XCOLS)

        def step(s, from_stage):
            lvec = leaf_v[pl.ds(s * _LANES, _LANES)]
            pvec = (part_v if from_stage else parx_v)[pl.ds(s * _LANES,
                                                            _LANES)]
            tcol = s * _LANES + lane
            pas, lfs = [], []
            for r in range(_R):
                if from_stage:
                    pas.append(plsc.load_gather(
                        stage, [pvec + jnp.int32(r * _NUM_TERMS)]))
                else:
                    pas.append(plsc.load_gather(
                        x_all, [pvec + (xoff + r * _XCOLS)]))
                lfs.append(plsc.load_gather(
                    x_all, [lvec + (xoff + r * _XCOLS)]))
            for r in range(_R):
                plsc.store_scatter(stage,
                                   [tcol + jnp.int32(r * _NUM_TERMS)],
                                   pas[r] * lfs[r])

        @plsc.parallel_loop(0, _MIXED_CHUNK, unroll=1)
        def _pa(s):
            step(s, False)

        s = _MIXED_CHUNK
        lvec = leaf_v[pl.ds(s * _LANES, _LANES)]
        pxv = parx_v[pl.ds(s * _LANES, _LANES)]
        ptv = part_v[pl.ds(s * _LANES, _LANES)]
        tcol = s * _LANES + lane
        vals = []
        for r in range(_R):
            pa_x = plsc.load_gather(x_all, [pxv + (xoff + r * _XCOLS)])
            pa_t = plsc.load_gather(stage, [ptv + jnp.int32(r * _NUM_TERMS)])
            lf = plsc.load_gather(x_all, [lvec + (xoff + r * _XCOLS)])
            vals.append(jnp.where(mixed, pa_t, pa_x) * lf)
        for r in range(_R):
            plsc.store_scatter(stage, [tcol + jnp.int32(r * _NUM_TERMS)],
                               vals[r])

        @plsc.parallel_loop(_MIXED_CHUNK + 1, _B_END, unroll=2)
        def _pb(s2):
            step(s2, True)

        @plsc.parallel_loop(_B_END, _TC_STEPS - 1, unroll=2)
        def _pc(s3):
            step(s3, True)

        s = _TC_STEPS - 1
        lvec = leaf_v[pl.ds(s * _LANES, _LANES)]
        ptv = part_v[pl.ds(s * _LANES, _LANES)]
        tcol = s * _LANES + lane
        vals = []
        for r in range(_R):
            pa = plsc.load_gather(stage, [ptv + jnp.int32(r * _NUM_TERMS)])
            lf = plsc.load_gather(x_all, [lvec + (xoff + r * _XCOLS)])
            vals.append(pa * lf)
        for r in range(_R):
            plsc.store_scatter(stage, [tcol + jnp.int32(r * _NUM_TERMS)],
                               vals[r], mask=tail_mask)

    def dst(sl):
        return out_hbm.at[pl.ds((wid * _ROWS_PER_W + sl * _R) * _NUM_TERMS,
                                _SLAB)]

    def pair_body(k, carry):
        sl0 = 2 * k
        sl1 = 2 * k + 1

        @pl.when(k > 0)
        def _w0():
            pltpu.make_async_copy(stage0, dst(sl0), sem0).wait()

        compute_slab(sl0, stage0)
        pltpu.async_copy(stage0, dst(sl0), sem0)

        @pl.when(k > 0)
        def _w1():
            pltpu.make_async_copy(stage1, dst(sl1), sem1).wait()

        compute_slab(sl1, stage1)
        pltpu.async_copy(stage1, dst(sl1), sem1)
        return carry

    lax.fori_loop(0, _NCHUNK // 2, pair_body, 0)
    pltpu.make_async_copy(stage0, dst(_NCHUNK - 2), sem0).wait()
    pltpu.make_async_copy(stage1, dst(_NCHUNK - 1), sem1).wait()


@functools.lru_cache(maxsize=1)
def _sc_call():
    return functools.partial(
        pl.kernel,
        out_type=jax.ShapeDtypeStruct((_BATCH * _NUM_TERMS,), jnp.float32),
        mesh=plsc.VectorSubcoreMesh(core_axis_name="c", subcore_axis_name="s"),
        scratch_types=[
            pltpu.VMEM((_TPAD,), jnp.int32),
            pltpu.VMEM((_TPAD,), jnp.int32),
            pltpu.VMEM((_TPAD,), jnp.int32),
            pltpu.VMEM((_ROWS_PER_W * _XCOLS,), jnp.float32),
            pltpu.VMEM((_SLAB,), jnp.float32),
            pltpu.VMEM((_SLAB,), jnp.float32),
            pltpu.SemaphoreType.DMA,
            pltpu.SemaphoreType.DMA,
        ],
        compiler_params=pltpu.CompilerParams(
            use_tc_tiling_on_sc=False, needs_layout_passes=False),
    )(_sc_parity)


@jax.jit
def kernel(inputs):
    batch = inputs.shape[0]
    x = jnp.concatenate(
        [inputs,
         jnp.ones((batch, 1), jnp.float32),
         jnp.zeros((batch, _XCOLS - _N_BITS - 1), jnp.float32)],
        axis=1).reshape(-1)
    flat = _sc_call()(x, jnp.asarray(_PARX_NP), jnp.asarray(_PART_NP),
                      jnp.asarray(_LEAF_NP))
    return flat.reshape(batch, _NUM_TERMS)


# speedup vs baseline: 3.1969x; 2.5120x over previous
import functools
import itertools

import jax
import jax.numpy as jnp
import numpy as np
from jax import lax
from jax.experimental import pallas as pl
from jax.experimental.pallas import tpu as pltpu
from jax.experimental.pallas import tpu_sc as plsc

_N_BITS = 20
_ORDER = 4
_BATCH = 4096
_NC = 2
_NS = 16
_NW = _NC * _NS
_ROWS_PER_W = _BATCH // _NW
_R = 8
_NCHUNK = _ROWS_PER_W // _R
_XCOLS = 24
_LANES = 16


def _build_tables():
    combos = []
    for size in range(1, _ORDER + 1):
        combos.extend(itertools.combinations(range(_N_BITS), size))
    num_terms = len(combos)
    tpad = ((num_terms + _LANES - 1) // _LANES) * _LANES
    par_x = np.full(tpad, 20, np.int32)
    par_t = np.zeros(tpad, np.int32)
    from_term = np.zeros(tpad, bool)
    leaf = np.full(tpad, 21, np.int32)
    idx_of = {}
    for t, c in enumerate(combos):
        idx_of[c] = t
        leaf[t] = c[-1]
        if len(c) == 1:
            par_x[t] = 20
        elif len(c) == 2:
            par_x[t] = c[0]
        else:
            from_term[t] = True
            par_t[t] = idx_of[c[:-1]]
    for t in range(num_terms):
        if from_term[t]:
            assert par_t[t] // _LANES < t // _LANES
    first_term_parent = int(np.argmax(from_term))
    mixed_chunk = first_term_parent // _LANES
    assert not from_term[: mixed_chunk * _LANES].any()
    mixed_lane = first_term_parent - mixed_chunk * _LANES
    assert from_term[first_term_parent: num_terms].all()
    return (par_x, par_t, leaf, mixed_lane, mixed_chunk, num_terms, tpad)


(_PARX_NP, _PART_NP, _LEAF_NP, _MIXED_LANE, _MIXED_CHUNK, _NUM_TERMS,
 _TPAD) = _build_tables()
_TC_STEPS = _TPAD // _LANES
_SLAB = _R * _NUM_TERMS
_B_END = 85


def _sc_parity(x_hbm, parx_hbm, part_hbm, leaf_hbm, out_hbm,
               parx_v, part_v, leaf_v, x_all, stage0, stage1, sem0, sem1):
    wid = lax.axis_index("s") * _NC + lax.axis_index("c")
    pltpu.sync_copy(parx_hbm, parx_v)
    pltpu.sync_copy(part_hbm, part_v)
    pltpu.sync_copy(leaf_hbm, leaf_v)
    pltpu.sync_copy(x_hbm.at[pl.ds(wid * _ROWS_PER_W, _ROWS_PER_W), :],
                    x_all)
    lane = lax.iota(jnp.int32, _LANES)
    mixed = lane >= jnp.int32(_MIXED_LANE)
    tail_mask = lane < jnp.int32(_NUM_TERMS - (_TC_STEPS - 1) * _LANES)

    def compute_slab(sl, stage):
        rbase = sl * _R

        def rsplat(r):
            return jnp.full((_LANES,), r, jnp.int32)

        def xrow(r):
            return rbase + rsplat(r)

        def step(s, from_stage):
            lvec = leaf_v[pl.ds(s * _LANES, _LANES)]
            pvec = (part_v if from_stage else parx_v)[pl.ds(s * _LANES,
                                                            _LANES)]
            tcol = s * _LANES + lane
            pas, lfs = [], []
            for r in range(_R):
                if from_stage:
                    pas.append(plsc.load_gather(stage, [rsplat(r), pvec]))
                else:
                    pas.append(plsc.load_gather(x_all, [xrow(r), pvec]))
                lfs.append(plsc.load_gather(x_all, [xrow(r), lvec]))
            for r in range(_R):
                plsc.store_scatter(stage, [rsplat(r), tcol],
                                   pas[r] * lfs[r])

        @plsc.parallel_loop(0, _MIXED_CHUNK, unroll=1)
        def _pa(s):
            step(s, False)

        s = _MIXED_CHUNK
        lvec = leaf_v[pl.ds(s * _LANES, _LANES)]
        pxv = parx_v[pl.ds(s * _LANES, _LANES)]
        ptv = part_v[pl.ds(s * _LANES, _LANES)]
        tcol = s * _LANES + lane
        vals = []
        for r in range(_R):
            pa_x = plsc.load_gather(x_all, [xrow(r), pxv])
            pa_t = plsc.load_gather(stage, [rsplat(r), ptv])
            lf = plsc.load_gather(x_all, [xrow(r), lvec])
            vals.append(jnp.where(mixed, pa_t, pa_x) * lf)
        for r in range(_R):
            plsc.store_scatter(stage, [rsplat(r), tcol], vals[r])

        @plsc.parallel_loop(_MIXED_CHUNK + 1, _B_END, unroll=2)
        def _pb(s2):
            step(s2, True)

        @plsc.parallel_loop(_B_END, _TC_STEPS - 1, unroll=2)
        def _pc(s3):
            step(s3, True)

        s = _TC_STEPS - 1
        lvec = leaf_v[pl.ds(s * _LANES, _LANES)]
        ptv = part_v[pl.ds(s * _LANES, _LANES)]
        tcol = s * _LANES + lane
        vals = []
        for r in range(_R):
            pa = plsc.load_gather(stage, [rsplat(r), ptv])
            lf = plsc.load_gather(x_all, [xrow(r), lvec])
            vals.append(pa * lf)
        for r in range(_R):
            plsc.store_scatter(stage, [rsplat(r), tcol], vals[r],
                               mask=tail_mask)

    def dst(sl):
        return out_hbm.at[pl.ds(wid * _ROWS_PER_W + sl * _R, _R), :]

    def pair_body(k, carry):
        sl0 = 2 * k
        sl1 = 2 * k + 1

        @pl.when(k > 0)
        def _w0():
            pltpu.make_async_copy(stage0, dst(sl0), sem0).wait()

        compute_slab(sl0, stage0)
        pltpu.async_copy(stage0, dst(sl0), sem0)

        @pl.when(k > 0)
        def _w1():
            pltpu.make_async_copy(stage1, dst(sl1), sem1).wait()

        compute_slab(sl1, stage1)
        pltpu.async_copy(stage1, dst(sl1), sem1)
        return carry

    lax.fori_loop(0, _NCHUNK // 2, pair_body, 0)
    pltpu.make_async_copy(stage0, dst(_NCHUNK - 2), sem0).wait()
    pltpu.make_async_copy(stage1, dst(_NCHUNK - 1), sem1).wait()


@functools.lru_cache(maxsize=1)
def _sc_call():
    return functools.partial(
        pl.kernel,
        out_type=jax.ShapeDtypeStruct((_BATCH, _NUM_TERMS), jnp.float32),
        mesh=plsc.VectorSubcoreMesh(core_axis_name="c", subcore_axis_name="s"),
        scratch_types=[
            pltpu.VMEM((_TPAD,), jnp.int32),
            pltpu.VMEM((_TPAD,), jnp.int32),
            pltpu.VMEM((_TPAD,), jnp.int32),
            pltpu.VMEM((_ROWS_PER_W, _XCOLS), jnp.float32),
            pltpu.VMEM((_R, _NUM_TERMS), jnp.float32),
            pltpu.VMEM((_R, _NUM_TERMS), jnp.float32),
            pltpu.SemaphoreType.DMA,
            pltpu.SemaphoreType.DMA,
        ],
        compiler_params=pltpu.CompilerParams(
            use_tc_tiling_on_sc=False, needs_layout_passes=False),
    )(_sc_parity)


@jax.jit
def kernel(inputs):
    batch = inputs.shape[0]
    x = jnp.concatenate(
        [inputs,
         jnp.ones((batch, 1), jnp.float32),
         jnp.zeros((batch, _XCOLS - _N_BITS - 1), jnp.float32)],
        axis=1)
    return _sc_call()(x, jnp.asarray(_PARX_NP), jnp.asarray(_PART_NP),
                      jnp.asarray(_LEAF_NP))
